# baseline (speedup 1.0000x reference)
import functools

import jax
import jax.numpy as jnp
from jax import lax
from jax.experimental import pallas as pl
from jax.experimental.pallas import tpu as pltpu
from jax.experimental.pallas import tpu_sc as plsc

_GDIM = 300
_DMODEL = 768
_VOCAB = 100000

_NC = 2
_NS = 16
_NW = _NC * _NS
_L = 16

_CHUNK = 64

_BV = 4096
_NBLK = (_VOCAB + _BV - 1) // _BV
_NBLK0 = 13
_SPLIT = _NBLK0 * _BV


def _sc_gather_bin_build(n_rows: int, width: int, lo: int, hi: int,
                         inplace: bool):
    b_per_w = n_rows // _NW
    assert n_rows % _NW == 0 and b_per_w % _CHUNK == 0
    n_chunks = b_per_w // _CHUNK
    n_groups = b_per_w // _L

    mesh = plsc.VectorSubcoreMesh(core_axis_name="c", subcore_axis_name="s")

    def body(table_hbm, idx_hbm, out_hbm, idx_v, cidx, cpos,
             buf0, buf1, sem0, sem1):
        wid = lax.axis_index("s") * _NC + lax.axis_index("c")
        base = wid * b_per_w
        pltpu.sync_copy(idx_hbm.at[pl.ds(base, b_per_w)], idx_v)

        lane = lax.iota(jnp.int32, _L)
        cnt = jnp.zeros((_L,), jnp.int32)
        for g in range(n_groups):
            v = idx_v[pl.ds(g * _L, _L)]
            m = (v >= lo) & (v < hi)
            pc = plsc.cumsum(jnp.where(m, 1, 0))
            dst = cnt + pc - 1
            plsc.store_scatter(cidx, [dst // _CHUNK, dst % _CHUNK],
                               v - lo, mask=m)
            plsc.store_scatter(cpos, [dst // _CHUNK, dst % _CHUNK],
                               base + g * _L + lane, mask=m)
            cnt = cnt + plsc.all_reduce_population_count(m)

        last = jnp.maximum(cnt - 1, 0)
        last_idx = plsc.load_gather(cidx, [last // _CHUNK, last % _CHUNK])
        last_pos = plsc.load_gather(cpos, [last // _CHUNK, last % _CHUNK])
        pair = 2 * _CHUNK
        rounded = ((cnt + pair - 1) // pair) * pair
        for k in range(pair // _L):
            f = cnt + k * _L + lane
            fm = f < rounded
            plsc.store_scatter(cidx, [f // _CHUNK, f % _CHUNK],
                               last_idx, mask=fm)
            plsc.store_scatter(cpos, [f // _CHUNK, f % _CHUNK],
                               last_pos, mask=fm)
        n_pairs = lax.reduce_max(rounded // pair, (0,))

        def pair_body(i, carry):
            c0 = 2 * i
            c1 = 2 * i + 1
            cp0 = pltpu.async_copy(table_hbm.at[cidx.at[c0]], buf0, sem0)
            cp1 = pltpu.async_copy(table_hbm.at[cidx.at[c1]], buf1, sem1)
            cp0.wait()
            pltpu.sync_copy(buf0, out_hbm.at[cpos.at[c0]])
            cp1.wait()
            pltpu.sync_copy(buf1, out_hbm.at[cpos.at[c1]])
            return carry

        lax.fori_loop(0, n_pairs, pair_body, 0)

    scratch = [
        pltpu.VMEM((b_per_w,), jnp.int32),
        pltpu.VMEM((n_chunks + 1, _CHUNK), jnp.int32),
        pltpu.VMEM((n_chunks + 1, _CHUNK), jnp.int32),
        pltpu.VMEM((_CHUNK, width), jnp.float32),
        pltpu.VMEM((_CHUNK, width), jnp.float32),
        pltpu.SemaphoreType.DMA,
        pltpu.SemaphoreType.DMA,
    ]
    cp = pltpu.CompilerParams(needs_layout_passes=False)
    if inplace:
        return pl.kernel(body, out_type=(), mesh=mesh, scratch_types=scratch,
                         compiler_params=cp)
    return pl.kernel(
        body,
        out_type=jax.ShapeDtypeStruct((n_rows, width), jnp.float32),
        mesh=mesh,
        scratch_types=scratch,
        compiler_params=cp,
    )


def _tc_project_slab_build(blk_lo: int, n_blk: int, rows: int):
    def body(tT_ref, w_ref, b_ref, o_ref):
        o_ref[...] = (
            jax.lax.dot_general(
                tT_ref[...].astype(jnp.bfloat16),
                w_ref[...].astype(jnp.bfloat16),
                dimension_numbers=(((0,), (0,)), ((), ())),
                preferred_element_type=jnp.float32)
            + b_ref[...]
        )

    return pl.pallas_call(
        body,
        grid=(n_blk,),
        in_specs=[
            pl.BlockSpec((_GDIM, _BV), lambda i: (0, i + blk_lo)),
            pl.BlockSpec((_GDIM, _DMODEL), lambda i: (0, 0)),
            pl.BlockSpec((1, _DMODEL), lambda i: (0, 0)),
        ],
        out_specs=pl.BlockSpec((_BV, _DMODEL), lambda i: (i, 0)),
        out_shape=jax.ShapeDtypeStruct((rows, _DMODEL), jnp.float32),
    )


def kernel(x, glove_table, W, b):
    batch, hist = x.shape
    n_rows = batch * hist
    idx = x.T.astype(jnp.int32).reshape(-1)
    table_t = glove_table.T
    b2 = b.reshape(1, _DMODEL)
    p0 = _tc_project_slab_build(0, _NBLK0, _SPLIT)(table_t, W, b2)
    p1 = _tc_project_slab_build(_NBLK0, _NBLK - _NBLK0, _VOCAB - _SPLIT)(
        table_t, W, b2)
    out0 = _sc_gather_bin_build(n_rows, _DMODEL, 0, _SPLIT, False)(p0, idx)
    out_ref = jax.new_ref(out0)
    _sc_gather_bin_build(n_rows, _DMODEL, _SPLIT, _VOCAB, True)(
        p1, idx, out_ref)
    out_tm = out_ref[...]
    return out_tm.reshape(hist, batch, _DMODEL).transpose(1, 0, 2)


# speedup vs baseline: 1.2816x; 1.2816x over previous
import functools

import jax
import jax.numpy as jnp
from jax import lax
from jax.experimental import pallas as pl
from jax.experimental.pallas import tpu as pltpu
from jax.experimental.pallas import tpu_sc as plsc

_GDIM = 300
_DMODEL = 768

_NC = 2
_NS = 16
_NW = _NC * _NS

_CHUNK = 32
_NBUF = 4


def _sc_gather_build(n_rows: int, width: int):
    b_per_w = n_rows // _NW
    assert n_rows % _NW == 0 and b_per_w % _CHUNK == 0
    n_chunks = b_per_w // _CHUNK

    mesh = plsc.VectorSubcoreMesh(core_axis_name="c", subcore_axis_name="s")

    @functools.partial(
        pl.kernel,
        mesh=mesh,
        out_type=jax.ShapeDtypeStruct((n_rows, width), jnp.float32),
        scratch_types=[
            pltpu.VMEM((b_per_w,), jnp.int32),
        ] + [pltpu.VMEM((_CHUNK, width), jnp.float32)] * _NBUF
          + [pltpu.SemaphoreType.DMA] * _NBUF,
    )
    def sc_gather(table_hbm, idx_hbm, out_hbm, idx_v, *bufs_sems):
        bufs = bufs_sems[:_NBUF]
        sems = bufs_sems[_NBUF:]
        wid = lax.axis_index("s") * _NC + lax.axis_index("c")
        base = wid * b_per_w
        pltpu.sync_copy(idx_hbm.at[pl.ds(base, b_per_w)], idx_v)
        copies = [None] * n_chunks

        def start(j):
            copies[j] = pltpu.async_copy(
                table_hbm.at[idx_v.at[pl.ds(j * _CHUNK, _CHUNK)]],
                bufs[j % _NBUF], sems[j % _NBUF])

        for j in range(min(_NBUF - 1, n_chunks)):
            start(j)
        for i in range(n_chunks):
            if i + _NBUF - 1 < n_chunks:
                start(i + _NBUF - 1)
            copies[i].wait()
            pltpu.sync_copy(
                bufs[i % _NBUF], out_hbm.at[pl.ds(base + i * _CHUNK, _CHUNK)])

    return sc_gather


def _tc_project_table_build(vocab: int, bv: int):
    grid = (vocab + bv - 1) // bv

    def body(tT_ref, w_ref, b_ref, o_ref):
        o_ref[...] = (
            jax.lax.dot_general(
                tT_ref[...].astype(jnp.bfloat16),
                w_ref[...].astype(jnp.bfloat16),
                dimension_numbers=(((0,), (0,)), ((), ())),
                preferred_element_type=jnp.float32)
            + b_ref[...]
        )

    return pl.pallas_call(
        body,
        grid=(grid,),
        in_specs=[
            pl.BlockSpec((_GDIM, bv), lambda i: (0, i)),
            pl.BlockSpec((_GDIM, _DMODEL), lambda i: (0, 0)),
            pl.BlockSpec((1, _DMODEL), lambda i: (0, 0)),
        ],
        out_specs=pl.BlockSpec((bv, _DMODEL), lambda i: (i, 0)),
        out_shape=jax.ShapeDtypeStruct((vocab, _DMODEL), jnp.float32),
    )


def kernel(x, glove_table, W, b):
    batch, hist = x.shape
    vocab = glove_table.shape[0]
    n_rows = batch * hist
    idx = x.T.astype(jnp.int32).reshape(-1)
    table_t = glove_table.T
    proj = _tc_project_table_build(vocab, 4096)(
        table_t, W, b.reshape(1, _DMODEL))
    out_tm = _sc_gather_build(n_rows, _DMODEL)(proj, idx)
    return out_tm.reshape(hist, batch, _DMODEL).transpose(1, 0, 2)
